# baseline (speedup 1.0000x reference)
import functools

import jax
import jax.numpy as jnp
from jax.experimental import pallas as pl


@functools.lru_cache(maxsize=None)
def _eps_const(n, e):
    return jax.random.normal(jax.random.key(42), (n, e), dtype=jnp.float32)


def _router_kernel(xa_ref, xb_ref, wr_ref, br_ref, wn_ref, bn_ref, eps_ref,
                   rout_ref, idx_ref):
    xa = xa_ref[...]
    xb = xb_ref[...]
    h = xa.shape[1]
    r = (jnp.dot(xa, wr_ref[:h], preferred_element_type=jnp.float32)
         + jnp.dot(xb, wr_ref[h:], preferred_element_type=jnp.float32)
         + br_ref[...])
    nl = (jnp.dot(xa, wn_ref[:h], preferred_element_type=jnp.float32)
          + jnp.dot(xb, wn_ref[h:], preferred_element_type=jnp.float32)
          + bn_ref[...])
    noisy = r + eps_ref[...] * jnp.logaddexp(nl, 0.0)

    lanes = jax.lax.broadcasted_iota(jnp.int32, noisy.shape, 1)
    m0 = jnp.max(noisy, axis=1, keepdims=True)
    i0 = jnp.min(jnp.where(noisy == m0, lanes, noisy.shape[1]), axis=1,
                 keepdims=True)
    masked = jnp.where(lanes == i0, -jnp.inf, noisy)
    m1 = jnp.max(masked, axis=1, keepdims=True)
    i1 = jnp.min(jnp.where(masked == m1, lanes, noisy.shape[1]), axis=1,
                 keepdims=True)

    d = jnp.exp(m1 - m0)
    p0 = 1.0 / (1.0 + d)
    p1 = d / (1.0 + d)
    rout_ref[...] = (jnp.where(lanes == i0, p0, 0.0)
                     + jnp.where(lanes == i1, p1, 0.0))
    idx_ref[...] = jnp.concatenate([i0, i1], axis=1)


def kernel(x, W_route, b_route, W_noise, b_noise):
    n, dim = x.shape
    e = W_route.shape[0]
    eps = _eps_const(n, e)
    blk = 4096
    out = pl.pallas_call(
        _router_kernel,
        grid=(n // blk,),
        in_specs=[
            pl.BlockSpec((blk, dim // 2), lambda i: (i, 0)),
            pl.BlockSpec((blk, dim // 2), lambda i: (i, 1)),
            pl.BlockSpec((dim, e), lambda i: (0, 0)),
            pl.BlockSpec((1, e), lambda i: (0, 0)),
            pl.BlockSpec((dim, e), lambda i: (0, 0)),
            pl.BlockSpec((1, e), lambda i: (0, 0)),
            pl.BlockSpec((blk, e), lambda i: (i, 0)),
        ],
        out_specs=(
            pl.BlockSpec((blk, e), lambda i: (i, 0)),
            pl.BlockSpec((blk, 2), lambda i: (i, 0)),
        ),
        out_shape=(
            jax.ShapeDtypeStruct((n, e), jnp.float32),
            jax.ShapeDtypeStruct((n, 2), jnp.int32),
        ),
    )(x, x, W_route.T, b_route.reshape(1, e), W_noise.T,
      b_noise.reshape(1, e), eps)
    return out


# speedup vs baseline: 2.6532x; 2.6532x over previous
import functools

import jax
import jax.numpy as jnp
from jax.experimental import pallas as pl


@functools.lru_cache(maxsize=None)
def _eps_const_t(n, e):
    return jax.random.normal(jax.random.key(42), (n, e), dtype=jnp.float32).T


def _router_kernel(x_ref, wc_ref, bc_ref, eps_ref, rout_ref, idx_ref):
    xb = x_ref[...]
    logits = jax.lax.dot_general(
        wc_ref[...], xb, (((1,), (1,)), ((), ())),
        preferred_element_type=jnp.float32) + bc_ref[...]
    e = eps_ref.shape[0]
    r = logits[:e]
    nl = logits[e:]
    noisy = r + eps_ref[...] * jnp.logaddexp(nl, 0.0)

    sub = jax.lax.broadcasted_iota(jnp.int32, noisy.shape, 0)
    m0 = jnp.max(noisy, axis=0, keepdims=True)
    i0 = jnp.min(jnp.where(noisy == m0, sub, e), axis=0, keepdims=True)
    masked = jnp.where(sub == i0, -jnp.inf, noisy)
    m1 = jnp.max(masked, axis=0, keepdims=True)
    i1 = jnp.min(jnp.where(masked == m1, sub, e), axis=0, keepdims=True)

    d = jnp.exp(m1 - m0)
    p0 = 1.0 / (1.0 + d)
    p1 = d / (1.0 + d)
    rout_t = (jnp.where(sub == i0, p0, 0.0) + jnp.where(sub == i1, p1, 0.0))
    idx_t = jnp.concatenate([i0, i1], axis=0)
    rout_ref[...] = rout_t.T
    idx_ref[...] = idx_t.T


def kernel(x, W_route, b_route, W_noise, b_noise):
    n, dim = x.shape
    e = W_route.shape[0]
    eps_t = _eps_const_t(n, e)
    wc = jnp.concatenate([W_route, W_noise], axis=0)
    bc = jnp.concatenate([b_route, b_noise]).reshape(2 * e, 1)
    blk = 4096
    out = pl.pallas_call(
        _router_kernel,
        grid=(n // blk,),
        in_specs=[
            pl.BlockSpec((blk, dim), lambda i: (i, 0)),
            pl.BlockSpec((2 * e, dim), lambda i: (0, 0)),
            pl.BlockSpec((2 * e, 1), lambda i: (0, 0)),
            pl.BlockSpec((e, blk), lambda i: (0, i)),
        ],
        out_specs=(
            pl.BlockSpec((blk, e), lambda i: (i, 0)),
            pl.BlockSpec((blk, 2), lambda i: (i, 0)),
        ),
        out_shape=(
            jax.ShapeDtypeStruct((n, e), jnp.float32),
            jax.ShapeDtypeStruct((n, 2), jnp.int32),
        ),
    )(x, wc, bc, eps_t)
    return out
